# Initial kernel scaffold; baseline (speedup 1.0000x reference)
import jax
import jax.numpy as jnp
from jax.experimental import pallas as pl


def kernel(x, edge_index, edge_type, params):
    raise NotImplementedError("write your pallas kernel here")


# speedup vs baseline: 11.8889x; 11.8889x over previous
import functools

import jax
import jax.numpy as jnp
from jax import lax
from jax.experimental import pallas as pl
from jax.experimental.pallas import tpu as pltpu
from jax.experimental.pallas import tpu_sc as plsc

N = 10000
E = 320000
D = 128
R = 8
RP = 16
NB = 4

NC = 2
NS = 16
NW = NC * NS
CHUNK = 128
NCHK = 80
EPW = NCHK * CHUNK
NPAD = N + 16
RPW = NPAD // NS
BN = 200


def _sc_counts_body(dst_ref, typ_ref, zc_ref, cnt_ref,
                    d2d, t2d, tbuf, ohbuf, cacc, sem):
    c = lax.axis_index("c")
    s = lax.axis_index("s")
    wid = c * NS + s

    pltpu.sync_copy(dst_ref.at[wid], d2d)
    pltpu.sync_copy(typ_ref.at[wid], t2d)

    pltpu.sync_copy(zc_ref.at[pl.ds(s * RPW, RPW)],
                    cacc.at[pl.ds(s * RPW, RPW)])
    plsc.subcore_barrier()

    ones = jnp.ones((16,), jnp.float32)
    zv = jnp.zeros((16,), jnp.float32)
    iota = lax.iota(jnp.int32, 16)

    def chunk(j, _):
        for g in range(CHUNK // 16):
            sl = pl.ds(g * 16, 16)
            tbuf[sl] = t2d[j, sl]

        def edge(e, _):
            tv = plsc.load_gather(tbuf, [jnp.full((16,), 0, jnp.int32) + e])
            ohbuf[e, :] = jnp.where(iota == tv, ones, zv)
            return 0
        lax.fori_loop(0, CHUNK, edge, 0)

        pltpu.sync_copy(ohbuf, cacc.at[d2d.at[j]], add=True)
        return 0
    lax.fori_loop(0, NCHK, chunk, 0)
    plsc.subcore_barrier()

    pltpu.sync_copy(cacc.at[pl.ds(s * RPW, RPW)],
                    cnt_ref.at[c, pl.ds(s * RPW, RPW)])


def _sc_agg_body(xw_ref, norm_ref, src_ref, dst_ref, typ_ref, z64_ref, p_ref,
                 sbuf, dbuf, tbuf, gbuf, ncbuf, nbuf, nrowbuf, rowbuf, zbuf,
                 acc, sem, sem2):
    c = lax.axis_index("c")
    s = lax.axis_index("s")
    wid = c * NS + s

    pltpu.sync_copy(z64_ref, zbuf)

    for k in range(RPW // 64):
        pltpu.sync_copy(zbuf, acc.at[pl.ds(s * RPW + k * 64, 64)])
    pltpu.sync_copy(zbuf.at[pl.ds(0, RPW % 64)],
                    acc.at[pl.ds(s * RPW + (RPW // 64) * 64, RPW % 64)])
    plsc.subcore_barrier()

    iota = lax.iota(jnp.int32, 16)

    def chunk(j, _):
        pltpu.sync_copy(src_ref.at[wid, j], sbuf)
        pltpu.sync_copy(dst_ref.at[wid, j], dbuf)
        pltpu.sync_copy(typ_ref.at[wid, j], tbuf)
        for g in range(CHUNK // 16):
            sl = pl.ds(g * 16, 16)
            gbuf[sl] = tbuf[sl] * N + sbuf[sl]
            ncbuf[sl] = jnp.minimum(dbuf[sl], N - 1)
        pltpu.async_copy(xw_ref.at[gbuf], rowbuf, sem).wait()
        pltpu.async_copy(norm_ref.at[ncbuf], nrowbuf, sem2).wait()
        for g in range(CHUNK // 16):
            sl = pl.ds(g * 16, 16)
            nbuf[sl] = plsc.load_gather(nrowbuf, [g * 16 + iota, tbuf[sl]])

        def edge(e, _):
            ns = plsc.load_gather(nbuf, [jnp.full((16,), 0, jnp.int32) + e])
            for q in range(D // 16):
                qs = pl.ds(q * 16, 16)
                rowbuf[e, qs] = rowbuf[e, qs] * ns
            return 0
        lax.fori_loop(0, CHUNK, edge, 0)

        pltpu.sync_copy(rowbuf, acc.at[dbuf], add=True)
        return 0
    lax.fori_loop(0, NCHK, chunk, 0)
    plsc.subcore_barrier()

    pltpu.sync_copy(acc.at[pl.ds(s * RPW, RPW)],
                    p_ref.at[c, pl.ds(s * RPW, RPW)])


_SC_PARAMS = pltpu.CompilerParams(use_tc_tiling_on_sc=False,
                                  needs_layout_passes=False)
_SC_MESH = dict(core_axis_name="c", subcore_axis_name="s")


def _make_sc_counts():
    return pl.kernel(
        _sc_counts_body,
        out_type=(jax.ShapeDtypeStruct((NC, NPAD, RP), jnp.float32),),
        mesh=plsc.VectorSubcoreMesh(**_SC_MESH),
        scratch_types=(
            pltpu.VMEM((NCHK, CHUNK), jnp.int32),
            pltpu.VMEM((NCHK, CHUNK), jnp.int32),
            pltpu.VMEM((CHUNK,), jnp.int32),
            pltpu.VMEM((CHUNK, RP), jnp.float32),
            pltpu.VMEM_SHARED((NPAD, RP), jnp.float32),
            pltpu.SemaphoreType.DMA,
        ),
        compiler_params=_SC_PARAMS,
    )


def _make_sc_agg():
    return pl.kernel(
        _sc_agg_body,
        out_type=(jax.ShapeDtypeStruct((NC, NPAD, D), jnp.float32),),
        mesh=plsc.VectorSubcoreMesh(**_SC_MESH),
        scratch_types=(
            pltpu.VMEM((CHUNK,), jnp.int32),
            pltpu.VMEM((CHUNK,), jnp.int32),
            pltpu.VMEM((CHUNK,), jnp.int32),
            pltpu.VMEM((CHUNK,), jnp.int32),
            pltpu.VMEM((CHUNK,), jnp.int32),
            pltpu.VMEM((CHUNK,), jnp.float32),
            pltpu.VMEM((CHUNK, RP), jnp.float32),
            pltpu.VMEM((CHUNK, D), jnp.float32),
            pltpu.VMEM((64, D), jnp.float32),
            pltpu.VMEM_SHARED((NPAD, D), jnp.float32),
            pltpu.SemaphoreType.DMA,
            pltpu.SemaphoreType.DMA,
        ),
        compiler_params=_SC_PARAMS,
    )


def _tc_k0_body(x_ref, cnt_ref, weight_ref, comp_ref, norm_ref, xw_ref):
    cnt = cnt_ref[0] + cnt_ref[1]
    norm_ref[...] = 1.0 / jnp.maximum(cnt, 1.0)
    w = jnp.dot(comp_ref[...], weight_ref[...].reshape(NB, D * D),
                preferred_element_type=jnp.float32).reshape(R, D, D)
    x = x_ref[...]
    for r in range(R):
        xw_ref[r] = jnp.dot(x, w[r], preferred_element_type=jnp.float32)


def _tc_k0(x, cnt, p0):
    fn = pl.pallas_call(
        _tc_k0_body,
        grid=(N // BN,),
        in_specs=[
            pl.BlockSpec((BN, D), lambda i: (i, 0)),
            pl.BlockSpec((NC, BN, RP), lambda i: (0, i, 0)),
            pl.BlockSpec((NB, D, D), lambda i: (0, 0, 0)),
            pl.BlockSpec((R, NB), lambda i: (0, 0)),
        ],
        out_specs=[
            pl.BlockSpec((BN, RP), lambda i: (i, 0)),
            pl.BlockSpec((R, BN, D), lambda i: (0, i, 0)),
        ],
        out_shape=[
            jax.ShapeDtypeStruct((N, RP), jnp.float32),
            jax.ShapeDtypeStruct((R, N, D), jnp.float32),
        ],
    )
    return fn(x, cnt, p0['weight'], p0['comp'])


def _tc_layer_body(outer_relu, last, x_ref, p_ref, root_ref, bias_ref,
                   eps_ref, w1_ref, b1_ref, g_ref, bln_ref, w2_ref, b2_ref,
                   *rest):
    if last:
        (h_ref,) = rest
    else:
        nweight_ref, ncomp_ref, h_ref, xw_ref = rest
    x = x_ref[...]
    pblk = p_ref[...]
    agg = pblk[0] + pblk[1]
    out = agg + jnp.dot(x, root_ref[...], preferred_element_type=jnp.float32)
    out = out + bias_ref[...] + (1.0 + eps_ref[0, 0]) * x
    h = jnp.dot(out, w1_ref[...], preferred_element_type=jnp.float32) + b1_ref[...]
    mu = jnp.mean(h, axis=-1, keepdims=True)
    var = jnp.mean((h - mu) ** 2, axis=-1, keepdims=True)
    h = (h - mu) * lax.rsqrt(var + 1e-5) * g_ref[...] + bln_ref[...]
    h = jnp.maximum(h, 0.0)
    h = jnp.dot(h, w2_ref[...], preferred_element_type=jnp.float32) + b2_ref[...]
    if outer_relu:
        h = jnp.maximum(h, 0.0)
    h_ref[...] = h
    if not last:
        w = jnp.dot(ncomp_ref[...], nweight_ref[...].reshape(NB, D * D),
                    preferred_element_type=jnp.float32).reshape(R, D, D)
        for r in range(R):
            xw_ref[r] = jnp.dot(h, w[r], preferred_element_type=jnp.float32)


def _tc_layer(x, pp, p, pnext):
    last = pnext is None
    in_specs = [
        pl.BlockSpec((BN, D), lambda i: (i, 0)),
        pl.BlockSpec((NC, BN, D), lambda i: (0, i, 0)),
        pl.BlockSpec((D, D), lambda i: (0, 0)),
        pl.BlockSpec((1, D), lambda i: (0, 0)),
        pl.BlockSpec((1, 1), lambda i: (0, 0)),
        pl.BlockSpec((D, D), lambda i: (0, 0)),
        pl.BlockSpec((1, D), lambda i: (0, 0)),
        pl.BlockSpec((1, D), lambda i: (0, 0)),
        pl.BlockSpec((1, D), lambda i: (0, 0)),
        pl.BlockSpec((D, D), lambda i: (0, 0)),
        pl.BlockSpec((1, D), lambda i: (0, 0)),
    ]
    args = [x, pp, p['root'], p['bias'].reshape(1, D), p['eps'].reshape(1, 1),
            p['W1'], p['b1'].reshape(1, D), p['g'].reshape(1, D),
            p['bln'].reshape(1, D), p['W2'], p['b2'].reshape(1, D)]
    out_specs = [pl.BlockSpec((BN, D), lambda i: (i, 0))]
    out_shape = [jax.ShapeDtypeStruct((N, D), jnp.float32)]
    if not last:
        in_specs += [pl.BlockSpec((NB, D, D), lambda i: (0, 0, 0)),
                     pl.BlockSpec((R, NB), lambda i: (0, 0))]
        args += [pnext['weight'], pnext['comp']]
        out_specs.append(pl.BlockSpec((R, BN, D), lambda i: (0, i, 0)))
        out_shape.append(jax.ShapeDtypeStruct((R, N, D), jnp.float32))
    fn = pl.pallas_call(
        functools.partial(_tc_layer_body, not last, last),
        grid=(N // BN,),
        in_specs=in_specs,
        out_specs=out_specs,
        out_shape=out_shape,
    )
    return fn(*args)


@jax.jit
def kernel(x, edge_index, edge_type, params):
    z64 = jnp.zeros((64, D), jnp.float32)
    zc = jnp.zeros((NPAD, RP), jnp.float32)

    npad_e = NW * EPW - E
    pad_i = jnp.arange(npad_e, dtype=jnp.int32)
    src = jnp.concatenate([edge_index[0], (pad_i * 197) % N]).reshape(NW, NCHK, CHUNK)
    dst = jnp.concatenate([edge_index[1], N + (pad_i % 16)]).reshape(NW, NCHK, CHUNK)
    et = jnp.concatenate([edge_type, pad_i % R]).reshape(NW, NCHK, CHUNK)

    sc_counts = _make_sc_counts()
    sc_agg = _make_sc_agg()

    (cnt,) = sc_counts(dst, et, zc)
    norm, xw1 = _tc_k0(x, cnt, params[0])
    (p1,) = sc_agg(xw1.reshape(R * N, D), norm, src, dst, et, z64)
    h1, xw2 = _tc_layer(x, p1, params[0], params[1])
    (p2,) = sc_agg(xw2.reshape(R * N, D), norm, src, dst, et, z64)
    h2, xw3 = _tc_layer(h1, p2, params[1], params[2])
    (p3,) = sc_agg(xw3.reshape(R * N, D), norm, src, dst, et, z64)
    (h3,) = _tc_layer(h2, p3, params[2], None)
    return h3


# speedup vs baseline: 20.1403x; 1.6940x over previous
import functools

import jax
import jax.numpy as jnp
from jax import lax
from jax.experimental import pallas as pl
from jax.experimental.pallas import tpu as pltpu
from jax.experimental.pallas import tpu_sc as plsc

N = 10000
E = 320000
D = 128
R = 8
RP = 16
NB = 4

NC = 2
NS = 16
NW = NC * NS
CHUNK = 128
NCHK = 80
EPW = NCHK * CHUNK
NPAD = N + 16
RPW = NPAD // NS
BN = 200


def _sc_counts_body(dst_ref, typ_ref, zc_ref, cnt_ref,
                    d2d, t2d, tbuf, ohbuf, cacc, sem):
    c = lax.axis_index("c")
    s = lax.axis_index("s")
    wid = c * NS + s

    pltpu.sync_copy(dst_ref.at[wid], d2d)
    pltpu.sync_copy(typ_ref.at[wid], t2d)

    pltpu.sync_copy(zc_ref.at[pl.ds(s * RPW, RPW)],
                    cacc.at[pl.ds(s * RPW, RPW)])
    plsc.subcore_barrier()

    ones = jnp.ones((16,), jnp.float32)
    zv = jnp.zeros((16,), jnp.float32)
    iota = lax.iota(jnp.int32, 16)

    def chunk(j, _):
        for g in range(CHUNK // 16):
            sl = pl.ds(g * 16, 16)
            tbuf[sl] = t2d[j, sl]

        def edge(e, _):
            tv = plsc.load_gather(tbuf, [jnp.full((16,), 0, jnp.int32) + e])
            ohbuf[e, :] = jnp.where(iota == tv, ones, zv)
            return 0
        lax.fori_loop(0, CHUNK, edge, 0)

        pltpu.sync_copy(ohbuf, cacc.at[d2d.at[j]], add=True)
        return 0
    lax.fori_loop(0, NCHK, chunk, 0)
    plsc.subcore_barrier()

    pltpu.sync_copy(cacc.at[pl.ds(s * RPW, RPW)],
                    cnt_ref.at[c, pl.ds(s * RPW, RPW)])


def _sc_agg_body(xw_ref, norm_ref, ei_ref, z64_ref, p_ref,
                 eb0, eb1, nb0, nb1, nr0, nr1, rb0, rb1, zbuf,
                 acc, sx0, sx1, sn0, sn1):
    c = lax.axis_index("c")
    s = lax.axis_index("s")
    wid = c * NS + s

    pltpu.sync_copy(z64_ref, zbuf)

    for k in range(RPW // 64):
        pltpu.sync_copy(zbuf, acc.at[pl.ds(s * RPW + k * 64, 64)])
    pltpu.sync_copy(zbuf.at[pl.ds(0, RPW % 64)],
                    acc.at[pl.ds(s * RPW + (RPW // 64) * 64, RPW % 64)])
    plsc.subcore_barrier()

    iota = lax.iota(jnp.int32, 16)
    ebs = (eb0, eb1)
    nbs = (nb0, nb1)
    nrs = (nr0, nr1)
    rbs = (rb0, rb1)
    sxs = (sx0, sx1)
    sns = (sn0, sn1)

    def fire(j, b):
        pltpu.sync_copy(ei_ref.at[wid, j], ebs[b])
        pltpu.async_copy(xw_ref.at[ebs[b].at[0]], rbs[b], sxs[b])
        pltpu.async_copy(norm_ref.at[ebs[b].at[2]], nrs[b], sns[b])

    def consume(b):
        eb, nb, nr, rb = ebs[b], nbs[b], nrs[b], rbs[b]
        pltpu.make_async_copy(xw_ref.at[pl.ds(0, CHUNK)], rb, sxs[b]).wait()
        pltpu.make_async_copy(norm_ref.at[pl.ds(0, CHUNK)], nr, sns[b]).wait()
        for g in range(CHUNK // 16):
            sl = pl.ds(g * 16, 16)
            nb[sl] = plsc.load_gather(nr, [g * 16 + iota, eb[3, sl]])

        def edge(e, _):
            ns = plsc.load_gather(nb, [jnp.full((16,), 0, jnp.int32) + e])
            for q in range(D // 16):
                qs = pl.ds(q * 16, 16)
                rb[e, qs] = rb[e, qs] * ns
            return 0
        lax.fori_loop(0, CHUNK, edge, 0)

        pltpu.sync_copy(rb, acc.at[eb.at[1]], add=True)

    fire(0, 0)

    def pair(i, _):
        j = 2 * i
        fire(j + 1, 1)
        consume(0)
        fire(jnp.minimum(j + 2, NCHK - 1), 0)
        consume(1)
        return 0
    lax.fori_loop(0, NCHK // 2, pair, 0)

    pltpu.make_async_copy(xw_ref.at[pl.ds(0, CHUNK)], rb0, sx0).wait()
    pltpu.make_async_copy(norm_ref.at[pl.ds(0, CHUNK)], nr0, sn0).wait()
    plsc.subcore_barrier()

    pltpu.sync_copy(acc.at[pl.ds(s * RPW, RPW)],
                    p_ref.at[c, pl.ds(s * RPW, RPW)])


_SC_PARAMS = pltpu.CompilerParams(use_tc_tiling_on_sc=False,
                                  needs_layout_passes=False)
_SC_MESH = dict(core_axis_name="c", subcore_axis_name="s")


def _make_sc_counts():
    return pl.kernel(
        _sc_counts_body,
        out_type=(jax.ShapeDtypeStruct((NC, NPAD, RP), jnp.float32),),
        mesh=plsc.VectorSubcoreMesh(**_SC_MESH),
        scratch_types=(
            pltpu.VMEM((NCHK, CHUNK), jnp.int32),
            pltpu.VMEM((NCHK, CHUNK), jnp.int32),
            pltpu.VMEM((CHUNK,), jnp.int32),
            pltpu.VMEM((CHUNK, RP), jnp.float32),
            pltpu.VMEM_SHARED((NPAD, RP), jnp.float32),
            pltpu.SemaphoreType.DMA,
        ),
        compiler_params=_SC_PARAMS,
    )


def _make_sc_agg():
    return pl.kernel(
        _sc_agg_body,
        out_type=(jax.ShapeDtypeStruct((NC, NPAD, D), jnp.float32),),
        mesh=plsc.VectorSubcoreMesh(**_SC_MESH),
        scratch_types=(
            pltpu.VMEM((4, CHUNK), jnp.int32),
            pltpu.VMEM((4, CHUNK), jnp.int32),
            pltpu.VMEM((CHUNK,), jnp.float32),
            pltpu.VMEM((CHUNK,), jnp.float32),
            pltpu.VMEM((CHUNK, RP), jnp.float32),
            pltpu.VMEM((CHUNK, RP), jnp.float32),
            pltpu.VMEM((CHUNK, D), jnp.float32),
            pltpu.VMEM((CHUNK, D), jnp.float32),
            pltpu.VMEM((64, D), jnp.float32),
            pltpu.VMEM_SHARED((NPAD, D), jnp.float32),
            pltpu.SemaphoreType.DMA,
            pltpu.SemaphoreType.DMA,
            pltpu.SemaphoreType.DMA,
            pltpu.SemaphoreType.DMA,
        ),
        compiler_params=_SC_PARAMS,
    )


def _tc_k0_body(x_ref, cnt_ref, weight_ref, comp_ref, norm_ref, xw_ref):
    cnt = cnt_ref[0] + cnt_ref[1]
    norm_ref[...] = 1.0 / jnp.maximum(cnt, 1.0)
    w = jnp.dot(comp_ref[...], weight_ref[...].reshape(NB, D * D),
                preferred_element_type=jnp.float32).reshape(R, D, D)
    x = x_ref[...]
    for r in range(R):
        xw_ref[r] = jnp.dot(x, w[r], preferred_element_type=jnp.float32)


def _tc_k0(x, cnt, p0):
    fn = pl.pallas_call(
        _tc_k0_body,
        grid=(N // BN,),
        in_specs=[
            pl.BlockSpec((BN, D), lambda i: (i, 0)),
            pl.BlockSpec((NC, BN, RP), lambda i: (0, i, 0)),
            pl.BlockSpec((NB, D, D), lambda i: (0, 0, 0)),
            pl.BlockSpec((R, NB), lambda i: (0, 0)),
        ],
        out_specs=[
            pl.BlockSpec((BN, RP), lambda i: (i, 0)),
            pl.BlockSpec((R, BN, D), lambda i: (0, i, 0)),
        ],
        out_shape=[
            jax.ShapeDtypeStruct((N, RP), jnp.float32),
            jax.ShapeDtypeStruct((R, N, D), jnp.float32),
        ],
    )
    return fn(x, cnt, p0['weight'], p0['comp'])


def _tc_layer_body(outer_relu, last, x_ref, p_ref, root_ref, bias_ref,
                   eps_ref, w1_ref, b1_ref, g_ref, bln_ref, w2_ref, b2_ref,
                   *rest):
    if last:
        (h_ref,) = rest
    else:
        nweight_ref, ncomp_ref, h_ref, xw_ref = rest
    x = x_ref[...]
    pblk = p_ref[...]
    agg = pblk[0] + pblk[1]
    out = agg + jnp.dot(x, root_ref[...], preferred_element_type=jnp.float32)
    out = out + bias_ref[...] + (1.0 + eps_ref[0, 0]) * x
    h = jnp.dot(out, w1_ref[...], preferred_element_type=jnp.float32) + b1_ref[...]
    mu = jnp.mean(h, axis=-1, keepdims=True)
    var = jnp.mean((h - mu) ** 2, axis=-1, keepdims=True)
    h = (h - mu) * lax.rsqrt(var + 1e-5) * g_ref[...] + bln_ref[...]
    h = jnp.maximum(h, 0.0)
    h = jnp.dot(h, w2_ref[...], preferred_element_type=jnp.float32) + b2_ref[...]
    if outer_relu:
        h = jnp.maximum(h, 0.0)
    h_ref[...] = h
    if not last:
        w = jnp.dot(ncomp_ref[...], nweight_ref[...].reshape(NB, D * D),
                    preferred_element_type=jnp.float32).reshape(R, D, D)
        for r in range(R):
            xw_ref[r] = jnp.dot(h, w[r], preferred_element_type=jnp.float32)


def _tc_layer(x, pp, p, pnext):
    last = pnext is None
    in_specs = [
        pl.BlockSpec((BN, D), lambda i: (i, 0)),
        pl.BlockSpec((NC, BN, D), lambda i: (0, i, 0)),
        pl.BlockSpec((D, D), lambda i: (0, 0)),
        pl.BlockSpec((1, D), lambda i: (0, 0)),
        pl.BlockSpec((1, 1), lambda i: (0, 0)),
        pl.BlockSpec((D, D), lambda i: (0, 0)),
        pl.BlockSpec((1, D), lambda i: (0, 0)),
        pl.BlockSpec((1, D), lambda i: (0, 0)),
        pl.BlockSpec((1, D), lambda i: (0, 0)),
        pl.BlockSpec((D, D), lambda i: (0, 0)),
        pl.BlockSpec((1, D), lambda i: (0, 0)),
    ]
    args = [x, pp, p['root'], p['bias'].reshape(1, D), p['eps'].reshape(1, 1),
            p['W1'], p['b1'].reshape(1, D), p['g'].reshape(1, D),
            p['bln'].reshape(1, D), p['W2'], p['b2'].reshape(1, D)]
    out_specs = [pl.BlockSpec((BN, D), lambda i: (i, 0))]
    out_shape = [jax.ShapeDtypeStruct((N, D), jnp.float32)]
    if not last:
        in_specs += [pl.BlockSpec((NB, D, D), lambda i: (0, 0, 0)),
                     pl.BlockSpec((R, NB), lambda i: (0, 0))]
        args += [pnext['weight'], pnext['comp']]
        out_specs.append(pl.BlockSpec((R, BN, D), lambda i: (0, i, 0)))
        out_shape.append(jax.ShapeDtypeStruct((R, N, D), jnp.float32))
    fn = pl.pallas_call(
        functools.partial(_tc_layer_body, not last, last),
        grid=(N // BN,),
        in_specs=in_specs,
        out_specs=out_specs,
        out_shape=out_shape,
    )
    return fn(*args)


@jax.jit
def kernel(x, edge_index, edge_type, params):
    z64 = jnp.zeros((64, D), jnp.float32)
    zc = jnp.zeros((NPAD, RP), jnp.float32)

    npad_e = NW * EPW - E
    pad_i = jnp.arange(npad_e, dtype=jnp.int32)
    src = jnp.concatenate([edge_index[0], (pad_i * 197) % N]).reshape(NW, NCHK, CHUNK)
    dst = jnp.concatenate([edge_index[1], N + (pad_i % 16)]).reshape(NW, NCHK, CHUNK)
    et = jnp.concatenate([edge_type, pad_i % R]).reshape(NW, NCHK, CHUNK)
    ei = jnp.stack([et * N + src, dst, jnp.minimum(dst, N - 1), et], axis=2)

    sc_counts = _make_sc_counts()
    sc_agg = _make_sc_agg()

    (cnt,) = sc_counts(dst, et, zc)
    norm, xw1 = _tc_k0(x, cnt, params[0])
    (p1,) = sc_agg(xw1.reshape(R * N, D), norm, ei, z64)
    h1, xw2 = _tc_layer(x, p1, params[0], params[1])
    (p2,) = sc_agg(xw2.reshape(R * N, D), norm, ei, z64)
    h2, xw3 = _tc_layer(h1, p2, params[1], params[2])
    (p3,) = sc_agg(xw3.reshape(R * N, D), norm, ei, z64)
    (h3,) = _tc_layer(h2, p3, params[2], None)
    return h3


# speedup vs baseline: 24.1969x; 1.2014x over previous
import functools

import jax
import jax.numpy as jnp
from jax import lax
from jax.experimental import pallas as pl
from jax.experimental.pallas import tpu as pltpu
from jax.experimental.pallas import tpu_sc as plsc

N = 10000
E = 320000
D = 128
R = 8
RP = 16
NB = 4

NC = 2
NS = 16
NW = NC * NS
CHUNK = 128
NCHK = 80
EPW = NCHK * CHUNK
NPAD = N + 16
RPW = NPAD // NS
BN = 200


def _sc_counts_body(dst_ref, typ_ref, zc_ref, cnt_ref,
                    d2d, t2d, tbuf, ohbuf, cacc, sem):
    c = lax.axis_index("c")
    s = lax.axis_index("s")
    wid = c * NS + s

    pltpu.sync_copy(dst_ref.at[wid], d2d)
    pltpu.sync_copy(typ_ref.at[wid], t2d)

    pltpu.sync_copy(zc_ref.at[pl.ds(s * RPW, RPW)],
                    cacc.at[pl.ds(s * RPW, RPW)])
    plsc.subcore_barrier()

    ones = jnp.ones((16,), jnp.float32)
    zv = jnp.zeros((16,), jnp.float32)
    iota = lax.iota(jnp.int32, 16)

    def chunk(j, _):
        for g in range(CHUNK // 16):
            sl = pl.ds(g * 16, 16)
            tbuf[sl] = t2d[j, sl]

        def edge(e, _):
            tv = plsc.load_gather(tbuf, [jnp.full((16,), 0, jnp.int32) + e])
            ohbuf[e, :] = jnp.where(iota == tv, ones, zv)
            return 0
        lax.fori_loop(0, CHUNK, edge, 0)

        pltpu.sync_copy(ohbuf, cacc.at[d2d.at[j]], add=True)
        return 0
    lax.fori_loop(0, NCHK, chunk, 0)
    plsc.subcore_barrier()

    pltpu.sync_copy(cacc.at[pl.ds(s * RPW, RPW)],
                    cnt_ref.at[c, pl.ds(s * RPW, RPW)])


def _sc_agg_body(xw_ref, norm_ref, ei_ref, z64_ref, p_ref,
                 eb0, eb1, db0, db1, nb0, nb1, nr0, nr1, rb0, rb1, zbuf,
                 acc, sx0, sx1, sn0, sn1, ss0, ss1):
    c = lax.axis_index("c")
    s = lax.axis_index("s")
    wid = c * NS + s

    pltpu.sync_copy(z64_ref, zbuf)

    for k in range(RPW // 64):
        pltpu.sync_copy(zbuf, acc.at[pl.ds(s * RPW + k * 64, 64)])
    pltpu.sync_copy(zbuf.at[pl.ds(0, RPW % 64)],
                    acc.at[pl.ds(s * RPW + (RPW // 64) * 64, RPW % 64)])
    plsc.subcore_barrier()

    iota = lax.iota(jnp.int32, 16)
    ebs = (eb0, eb1)
    dbs = (db0, db1)
    nbs = (nb0, nb1)
    nrs = (nr0, nr1)
    rbs = (rb0, rb1)
    sxs = (sx0, sx1)
    sns = (sn0, sn1)
    sss = (ss0, ss1)

    def fire_ei(j, b):
        pltpu.sync_copy(ei_ref.at[wid, j], ebs[b])

    def fire(b):
        pltpu.async_copy(xw_ref.at[ebs[b].at[0]], rbs[b], sxs[b])
        pltpu.async_copy(norm_ref.at[ebs[b].at[2]], nrs[b], sns[b])

    def scale(b):
        eb, nb, nr, rb = ebs[b], nbs[b], nrs[b], rbs[b]
        pltpu.make_async_copy(xw_ref.at[pl.ds(0, CHUNK)], rb, sxs[b]).wait()
        pltpu.make_async_copy(norm_ref.at[pl.ds(0, CHUNK)], nr, sns[b]).wait()
        for g in range(CHUNK // 16):
            sl = pl.ds(g * 16, 16)
            nb[sl] = plsc.load_gather(nr, [g * 16 + iota, eb[3, sl]])

        @plsc.parallel_loop(0, CHUNK, unroll=4)
        def _(e):
            ns = plsc.load_gather(nb, [jnp.full((16,), 0, jnp.int32) + e])
            for q in range(D // 16):
                qs = pl.ds(q * 16, 16)
                rb[e, qs] = rb[e, qs] * ns

    def scat(b):
        for g in range(CHUNK // 16):
            sl = pl.ds(g * 16, 16)
            dbs[b][sl] = ebs[b][1, sl]
        pltpu.async_copy(rbs[b], acc.at[dbs[b]], sss[b], add=True)

    def wait_scat(b):
        pltpu.make_async_copy(rbs[b], acc.at[pl.ds(0, CHUNK)], sss[b]).wait()

    fire_ei(0, 0)
    fire(0)
    fire_ei(1, 1)
    fire(1)

    def pair(i, _):
        j = 2 * i
        scale(0)
        scat(0)
        fire_ei(jnp.minimum(j + 2, NCHK - 1), 0)
        scale(1)
        wait_scat(0)
        fire(0)
        scat(1)
        fire_ei(jnp.minimum(j + 3, NCHK - 1), 1)
        wait_scat(1)
        fire(1)
        return 0
    lax.fori_loop(0, NCHK // 2, pair, 0)

    pltpu.make_async_copy(xw_ref.at[pl.ds(0, CHUNK)], rb0, sx0).wait()
    pltpu.make_async_copy(norm_ref.at[pl.ds(0, CHUNK)], nr0, sn0).wait()
    pltpu.make_async_copy(xw_ref.at[pl.ds(0, CHUNK)], rb1, sx1).wait()
    pltpu.make_async_copy(norm_ref.at[pl.ds(0, CHUNK)], nr1, sn1).wait()
    plsc.subcore_barrier()

    pltpu.sync_copy(acc.at[pl.ds(s * RPW, RPW)],
                    p_ref.at[c, pl.ds(s * RPW, RPW)])


_SC_PARAMS = pltpu.CompilerParams(use_tc_tiling_on_sc=False,
                                  needs_layout_passes=False)
_SC_MESH = dict(core_axis_name="c", subcore_axis_name="s")


def _make_sc_counts():
    return pl.kernel(
        _sc_counts_body,
        out_type=(jax.ShapeDtypeStruct((NC, NPAD, RP), jnp.float32),),
        mesh=plsc.VectorSubcoreMesh(**_SC_MESH),
        scratch_types=(
            pltpu.VMEM((NCHK, CHUNK), jnp.int32),
            pltpu.VMEM((NCHK, CHUNK), jnp.int32),
            pltpu.VMEM((CHUNK,), jnp.int32),
            pltpu.VMEM((CHUNK, RP), jnp.float32),
            pltpu.VMEM_SHARED((NPAD, RP), jnp.float32),
            pltpu.SemaphoreType.DMA,
        ),
        compiler_params=_SC_PARAMS,
    )


def _make_sc_agg():
    return pl.kernel(
        _sc_agg_body,
        out_type=(jax.ShapeDtypeStruct((NC, NPAD, D), jnp.float32),),
        mesh=plsc.VectorSubcoreMesh(**_SC_MESH),
        scratch_types=(
            pltpu.VMEM((4, CHUNK), jnp.int32),
            pltpu.VMEM((4, CHUNK), jnp.int32),
            pltpu.VMEM((CHUNK,), jnp.int32),
            pltpu.VMEM((CHUNK,), jnp.int32),
            pltpu.VMEM((CHUNK,), jnp.float32),
            pltpu.VMEM((CHUNK,), jnp.float32),
            pltpu.VMEM((CHUNK, RP), jnp.float32),
            pltpu.VMEM((CHUNK, RP), jnp.float32),
            pltpu.VMEM((CHUNK, D), jnp.float32),
            pltpu.VMEM((CHUNK, D), jnp.float32),
            pltpu.VMEM((64, D), jnp.float32),
            pltpu.VMEM_SHARED((NPAD, D), jnp.float32),
            pltpu.SemaphoreType.DMA,
            pltpu.SemaphoreType.DMA,
            pltpu.SemaphoreType.DMA,
            pltpu.SemaphoreType.DMA,
            pltpu.SemaphoreType.DMA,
            pltpu.SemaphoreType.DMA,
        ),
        compiler_params=_SC_PARAMS,
    )


def _tc_k0_body(x_ref, cnt_ref, weight_ref, comp_ref, norm_ref, xw_ref):
    cnt = cnt_ref[0] + cnt_ref[1]
    norm_ref[...] = 1.0 / jnp.maximum(cnt, 1.0)
    w = jnp.dot(comp_ref[...], weight_ref[...].reshape(NB, D * D),
                preferred_element_type=jnp.float32).reshape(R, D, D)
    x = x_ref[...]
    for r in range(R):
        xw_ref[r] = jnp.dot(x, w[r], preferred_element_type=jnp.float32)


def _tc_k0(x, cnt, p0):
    fn = pl.pallas_call(
        _tc_k0_body,
        grid=(N // BN,),
        in_specs=[
            pl.BlockSpec((BN, D), lambda i: (i, 0)),
            pl.BlockSpec((NC, BN, RP), lambda i: (0, i, 0)),
            pl.BlockSpec((NB, D, D), lambda i: (0, 0, 0)),
            pl.BlockSpec((R, NB), lambda i: (0, 0)),
        ],
        out_specs=[
            pl.BlockSpec((BN, RP), lambda i: (i, 0)),
            pl.BlockSpec((R, BN, D), lambda i: (0, i, 0)),
        ],
        out_shape=[
            jax.ShapeDtypeStruct((N, RP), jnp.float32),
            jax.ShapeDtypeStruct((R, N, D), jnp.float32),
        ],
    )
    return fn(x, cnt, p0['weight'], p0['comp'])


def _tc_layer_body(outer_relu, last, x_ref, p_ref, root_ref, bias_ref,
                   eps_ref, w1_ref, b1_ref, g_ref, bln_ref, w2_ref, b2_ref,
                   *rest):
    if last:
        (h_ref,) = rest
    else:
        nweight_ref, ncomp_ref, h_ref, xw_ref = rest
    x = x_ref[...]
    pblk = p_ref[...]
    agg = pblk[0] + pblk[1]
    out = agg + jnp.dot(x, root_ref[...], preferred_element_type=jnp.float32)
    out = out + bias_ref[...] + (1.0 + eps_ref[0, 0]) * x
    h = jnp.dot(out, w1_ref[...], preferred_element_type=jnp.float32) + b1_ref[...]
    mu = jnp.mean(h, axis=-1, keepdims=True)
    var = jnp.mean((h - mu) ** 2, axis=-1, keepdims=True)
    h = (h - mu) * lax.rsqrt(var + 1e-5) * g_ref[...] + bln_ref[...]
    h = jnp.maximum(h, 0.0)
    h = jnp.dot(h, w2_ref[...], preferred_element_type=jnp.float32) + b2_ref[...]
    if outer_relu:
        h = jnp.maximum(h, 0.0)
    h_ref[...] = h
    if not last:
        w = jnp.dot(ncomp_ref[...], nweight_ref[...].reshape(NB, D * D),
                    preferred_element_type=jnp.float32).reshape(R, D, D)
        for r in range(R):
            xw_ref[r] = jnp.dot(h, w[r], preferred_element_type=jnp.float32)


def _tc_layer(x, pp, p, pnext):
    last = pnext is None
    in_specs = [
        pl.BlockSpec((BN, D), lambda i: (i, 0)),
        pl.BlockSpec((NC, BN, D), lambda i: (0, i, 0)),
        pl.BlockSpec((D, D), lambda i: (0, 0)),
        pl.BlockSpec((1, D), lambda i: (0, 0)),
        pl.BlockSpec((1, 1), lambda i: (0, 0)),
        pl.BlockSpec((D, D), lambda i: (0, 0)),
        pl.BlockSpec((1, D), lambda i: (0, 0)),
        pl.BlockSpec((1, D), lambda i: (0, 0)),
        pl.BlockSpec((1, D), lambda i: (0, 0)),
        pl.BlockSpec((D, D), lambda i: (0, 0)),
        pl.BlockSpec((1, D), lambda i: (0, 0)),
    ]
    args = [x, pp, p['root'], p['bias'].reshape(1, D), p['eps'].reshape(1, 1),
            p['W1'], p['b1'].reshape(1, D), p['g'].reshape(1, D),
            p['bln'].reshape(1, D), p['W2'], p['b2'].reshape(1, D)]
    out_specs = [pl.BlockSpec((BN, D), lambda i: (i, 0))]
    out_shape = [jax.ShapeDtypeStruct((N, D), jnp.float32)]
    if not last:
        in_specs += [pl.BlockSpec((NB, D, D), lambda i: (0, 0, 0)),
                     pl.BlockSpec((R, NB), lambda i: (0, 0))]
        args += [pnext['weight'], pnext['comp']]
        out_specs.append(pl.BlockSpec((R, BN, D), lambda i: (0, i, 0)))
        out_shape.append(jax.ShapeDtypeStruct((R, N, D), jnp.float32))
    fn = pl.pallas_call(
        functools.partial(_tc_layer_body, not last, last),
        grid=(N // BN,),
        in_specs=in_specs,
        out_specs=out_specs,
        out_shape=out_shape,
    )
    return fn(*args)


@jax.jit
def kernel(x, edge_index, edge_type, params):
    z64 = jnp.zeros((64, D), jnp.float32)
    zc = jnp.zeros((NPAD, RP), jnp.float32)

    npad_e = NW * EPW - E
    pad_i = jnp.arange(npad_e, dtype=jnp.int32)
    src = jnp.concatenate([edge_index[0], (pad_i * 197) % N]).reshape(NW, NCHK, CHUNK)
    dst = jnp.concatenate([edge_index[1], N + (pad_i % 16)]).reshape(NW, NCHK, CHUNK)
    et = jnp.concatenate([edge_type, pad_i % R]).reshape(NW, NCHK, CHUNK)
    ei = jnp.stack([et * N + src, dst, jnp.minimum(dst, N - 1), et], axis=2)

    sc_counts = _make_sc_counts()
    sc_agg = _make_sc_agg()

    (cnt,) = sc_counts(dst, et, zc)
    norm, xw1 = _tc_k0(x, cnt, params[0])
    (p1,) = sc_agg(xw1.reshape(R * N, D), norm, ei, z64)
    h1, xw2 = _tc_layer(x, p1, params[0], params[1])
    (p2,) = sc_agg(xw2.reshape(R * N, D), norm, ei, z64)
    h2, xw3 = _tc_layer(h1, p2, params[1], params[2])
    (p3,) = sc_agg(xw3.reshape(R * N, D), norm, ei, z64)
    (h3,) = _tc_layer(h2, p3, params[2], None)
    return h3


# speedup vs baseline: 27.9470x; 1.1550x over previous
import functools

import jax
import jax.numpy as jnp
from jax import lax
from jax.experimental import pallas as pl
from jax.experimental.pallas import tpu as pltpu
from jax.experimental.pallas import tpu_sc as plsc

N = 10000
E = 320000
D = 128
R = 8
RP = 16
NB = 4

NC = 2
NS = 16
NW = NC * NS
CHUNK = 128
NCHK = 80
EPW = NCHK * CHUNK
NPAD = N + 16
RPW = NPAD // NS
BN = 1000


def _sc_counts_body(dst_ref, typ_ref, zc_ref, cnt_ref,
                    d2d, t2d, tbuf, ohbuf, cacc, sem):
    c = lax.axis_index("c")
    s = lax.axis_index("s")
    wid = c * NS + s

    pltpu.sync_copy(dst_ref.at[wid], d2d)
    pltpu.sync_copy(typ_ref.at[wid], t2d)

    pltpu.sync_copy(zc_ref.at[pl.ds(s * RPW, RPW)],
                    cacc.at[pl.ds(s * RPW, RPW)])
    plsc.subcore_barrier()

    ones = jnp.ones((16,), jnp.float32)
    zv = jnp.zeros((16,), jnp.float32)
    iota = lax.iota(jnp.int32, 16)

    def chunk(j, _):
        for g in range(CHUNK // 16):
            sl = pl.ds(g * 16, 16)
            tbuf[sl] = t2d[j, sl]

        def edge(e, _):
            tv = plsc.load_gather(tbuf, [jnp.full((16,), 0, jnp.int32) + e])
            ohbuf[e, :] = jnp.where(iota == tv, ones, zv)
            return 0
        lax.fori_loop(0, CHUNK, edge, 0)

        pltpu.sync_copy(ohbuf, cacc.at[d2d.at[j]], add=True)
        return 0
    lax.fori_loop(0, NCHK, chunk, 0)
    plsc.subcore_barrier()

    pltpu.sync_copy(cacc.at[pl.ds(s * RPW, RPW)],
                    cnt_ref.at[c, pl.ds(s * RPW, RPW)])


def _sc_agg_body(xw_ref, norm_ref, ei_ref, z64_ref, p_ref,
                 eb0, eb1, db0, db1, nb0, nb1, nr0, nr1, rb0, rb1, zbuf,
                 acc, sx0, sx1, sn0, sn1, ss0, ss1):
    c = lax.axis_index("c")
    s = lax.axis_index("s")
    wid = c * NS + s

    pltpu.sync_copy(z64_ref, zbuf)

    for k in range(RPW // 64):
        pltpu.sync_copy(zbuf, acc.at[pl.ds(s * RPW + k * 64, 64)])
    pltpu.sync_copy(zbuf.at[pl.ds(0, RPW % 64)],
                    acc.at[pl.ds(s * RPW + (RPW // 64) * 64, RPW % 64)])
    plsc.subcore_barrier()

    iota = lax.iota(jnp.int32, 16)
    ebs = (eb0, eb1)
    dbs = (db0, db1)
    nbs = (nb0, nb1)
    nrs = (nr0, nr1)
    rbs = (rb0, rb1)
    sxs = (sx0, sx1)
    sns = (sn0, sn1)
    sss = (ss0, ss1)

    def fire_ei(j, b):
        pltpu.sync_copy(ei_ref.at[wid, j], ebs[b])

    def fire(b):
        pltpu.async_copy(xw_ref.at[ebs[b].at[0]], rbs[b], sxs[b])
        pltpu.async_copy(norm_ref.at[ebs[b].at[2]], nrs[b], sns[b])

    def scale(b):
        eb, nb, nr, rb = ebs[b], nbs[b], nrs[b], rbs[b]
        pltpu.make_async_copy(xw_ref.at[pl.ds(0, CHUNK)], rb, sxs[b]).wait()
        pltpu.make_async_copy(norm_ref.at[pl.ds(0, CHUNK)], nr, sns[b]).wait()
        for g in range(CHUNK // 16):
            sl = pl.ds(g * 16, 16)
            nb[sl] = plsc.load_gather(nr, [g * 16 + iota, eb[3, sl]])

        @plsc.parallel_loop(0, CHUNK, unroll=4)
        def _(e):
            ns = plsc.load_gather(nb, [jnp.full((16,), 0, jnp.int32) + e])
            for q in range(D // 16):
                qs = pl.ds(q * 16, 16)
                rb[e, qs] = rb[e, qs] * ns

    def scat(b):
        for g in range(CHUNK // 16):
            sl = pl.ds(g * 16, 16)
            dbs[b][sl] = ebs[b][1, sl]
        pltpu.async_copy(rbs[b], acc.at[dbs[b]], sss[b], add=True)

    def wait_scat(b):
        pltpu.make_async_copy(rbs[b], acc.at[pl.ds(0, CHUNK)], sss[b]).wait()

    fire_ei(0, 0)
    fire(0)
    fire_ei(1, 1)
    fire(1)

    def pair(i, _):
        j = 2 * i
        scale(0)
        scat(0)
        fire_ei(jnp.minimum(j + 2, NCHK - 1), 0)
        scale(1)
        wait_scat(0)
        fire(0)
        scat(1)
        fire_ei(jnp.minimum(j + 3, NCHK - 1), 1)
        wait_scat(1)
        fire(1)
        return 0
    lax.fori_loop(0, NCHK // 2, pair, 0)

    pltpu.make_async_copy(xw_ref.at[pl.ds(0, CHUNK)], rb0, sx0).wait()
    pltpu.make_async_copy(norm_ref.at[pl.ds(0, CHUNK)], nr0, sn0).wait()
    pltpu.make_async_copy(xw_ref.at[pl.ds(0, CHUNK)], rb1, sx1).wait()
    pltpu.make_async_copy(norm_ref.at[pl.ds(0, CHUNK)], nr1, sn1).wait()
    plsc.subcore_barrier()

    pltpu.sync_copy(acc.at[pl.ds(s * RPW, RPW)],
                    p_ref.at[c, pl.ds(s * RPW, RPW)])


_SC_PARAMS = pltpu.CompilerParams(use_tc_tiling_on_sc=False,
                                  needs_layout_passes=False)
_SC_MESH = dict(core_axis_name="c", subcore_axis_name="s")


def _make_sc_counts():
    return pl.kernel(
        _sc_counts_body,
        out_type=(jax.ShapeDtypeStruct((NC, NPAD, RP), jnp.float32),),
        mesh=plsc.VectorSubcoreMesh(**_SC_MESH),
        scratch_types=(
            pltpu.VMEM((NCHK, CHUNK), jnp.int32),
            pltpu.VMEM((NCHK, CHUNK), jnp.int32),
            pltpu.VMEM((CHUNK,), jnp.int32),
            pltpu.VMEM((CHUNK, RP), jnp.float32),
            pltpu.VMEM_SHARED((NPAD, RP), jnp.float32),
            pltpu.SemaphoreType.DMA,
        ),
        compiler_params=_SC_PARAMS,
    )


def _make_sc_agg():
    return pl.kernel(
        _sc_agg_body,
        out_type=(jax.ShapeDtypeStruct((NC, NPAD, D), jnp.float32),),
        mesh=plsc.VectorSubcoreMesh(**_SC_MESH),
        scratch_types=(
            pltpu.VMEM((4, CHUNK), jnp.int32),
            pltpu.VMEM((4, CHUNK), jnp.int32),
            pltpu.VMEM((CHUNK,), jnp.int32),
            pltpu.VMEM((CHUNK,), jnp.int32),
            pltpu.VMEM((CHUNK,), jnp.float32),
            pltpu.VMEM((CHUNK,), jnp.float32),
            pltpu.VMEM((CHUNK, RP), jnp.float32),
            pltpu.VMEM((CHUNK, RP), jnp.float32),
            pltpu.VMEM((CHUNK, D), jnp.float32),
            pltpu.VMEM((CHUNK, D), jnp.float32),
            pltpu.VMEM((64, D), jnp.float32),
            pltpu.VMEM_SHARED((NPAD, D), jnp.float32),
            pltpu.SemaphoreType.DMA,
            pltpu.SemaphoreType.DMA,
            pltpu.SemaphoreType.DMA,
            pltpu.SemaphoreType.DMA,
            pltpu.SemaphoreType.DMA,
            pltpu.SemaphoreType.DMA,
        ),
        compiler_params=_SC_PARAMS,
    )


def _tc_xw_body(x_ref, weight_ref, comp_ref, xw_ref):
    w = jnp.dot(comp_ref[...], weight_ref[...].reshape(NB, D * D),
                preferred_element_type=jnp.float32).reshape(R, D, D)
    x = x_ref[...]
    for r in range(R):
        xw_ref[r] = jnp.dot(x, w[r], preferred_element_type=jnp.float32)


def _tc_xw(x, p0):
    fn = pl.pallas_call(
        _tc_xw_body,
        grid=(N // BN,),
        in_specs=[
            pl.BlockSpec((BN, D), lambda i: (i, 0)),
            pl.BlockSpec((NB, D, D), lambda i: (0, 0, 0)),
            pl.BlockSpec((R, NB), lambda i: (0, 0)),
        ],
        out_specs=pl.BlockSpec((R, BN, D), lambda i: (0, i, 0)),
        out_shape=jax.ShapeDtypeStruct((R, N, D), jnp.float32),
    )
    return fn(x, p0['weight'], p0['comp'])


def _tc_norm_body(cnt_ref, norm_ref):
    cnt = cnt_ref[0] + cnt_ref[1]
    norm_ref[...] = 1.0 / jnp.maximum(cnt, 1.0)


def _tc_norm(cnt):
    fn = pl.pallas_call(
        _tc_norm_body,
        grid=(N // BN,),
        in_specs=[pl.BlockSpec((NC, BN, RP), lambda i: (0, i, 0))],
        out_specs=pl.BlockSpec((BN, RP), lambda i: (i, 0)),
        out_shape=jax.ShapeDtypeStruct((N, RP), jnp.float32),
    )
    return fn(cnt)


def _tc_layer_body(outer_relu, last, x_ref, p_ref, root_ref, bias_ref,
                   eps_ref, w1_ref, b1_ref, g_ref, bln_ref, w2_ref, b2_ref,
                   *rest):
    if last:
        (h_ref,) = rest
    else:
        nweight_ref, ncomp_ref, h_ref, xw_ref = rest
    x = x_ref[...]
    pblk = p_ref[...]
    agg = pblk[0] + pblk[1]
    out = agg + jnp.dot(x, root_ref[...], preferred_element_type=jnp.float32)
    out = out + bias_ref[...] + (1.0 + eps_ref[0, 0]) * x
    h = jnp.dot(out, w1_ref[...], preferred_element_type=jnp.float32) + b1_ref[...]
    mu = jnp.mean(h, axis=-1, keepdims=True)
    var = jnp.mean((h - mu) ** 2, axis=-1, keepdims=True)
    h = (h - mu) * lax.rsqrt(var + 1e-5) * g_ref[...] + bln_ref[...]
    h = jnp.maximum(h, 0.0)
    h = jnp.dot(h, w2_ref[...], preferred_element_type=jnp.float32) + b2_ref[...]
    if outer_relu:
        h = jnp.maximum(h, 0.0)
    h_ref[...] = h
    if not last:
        w = jnp.dot(ncomp_ref[...], nweight_ref[...].reshape(NB, D * D),
                    preferred_element_type=jnp.float32).reshape(R, D, D)
        for r in range(R):
            xw_ref[r] = jnp.dot(h, w[r], preferred_element_type=jnp.float32)


def _tc_layer(x, pp, p, pnext):
    last = pnext is None
    in_specs = [
        pl.BlockSpec((BN, D), lambda i: (i, 0)),
        pl.BlockSpec((NC, BN, D), lambda i: (0, i, 0)),
        pl.BlockSpec((D, D), lambda i: (0, 0)),
        pl.BlockSpec((1, D), lambda i: (0, 0)),
        pl.BlockSpec((1, 1), lambda i: (0, 0)),
        pl.BlockSpec((D, D), lambda i: (0, 0)),
        pl.BlockSpec((1, D), lambda i: (0, 0)),
        pl.BlockSpec((1, D), lambda i: (0, 0)),
        pl.BlockSpec((1, D), lambda i: (0, 0)),
        pl.BlockSpec((D, D), lambda i: (0, 0)),
        pl.BlockSpec((1, D), lambda i: (0, 0)),
    ]
    args = [x, pp, p['root'], p['bias'].reshape(1, D), p['eps'].reshape(1, 1),
            p['W1'], p['b1'].reshape(1, D), p['g'].reshape(1, D),
            p['bln'].reshape(1, D), p['W2'], p['b2'].reshape(1, D)]
    out_specs = [pl.BlockSpec((BN, D), lambda i: (i, 0))]
    out_shape = [jax.ShapeDtypeStruct((N, D), jnp.float32)]
    if not last:
        in_specs += [pl.BlockSpec((NB, D, D), lambda i: (0, 0, 0)),
                     pl.BlockSpec((R, NB), lambda i: (0, 0))]
        args += [pnext['weight'], pnext['comp']]
        out_specs.append(pl.BlockSpec((R, BN, D), lambda i: (0, i, 0)))
        out_shape.append(jax.ShapeDtypeStruct((R, N, D), jnp.float32))
    fn = pl.pallas_call(
        functools.partial(_tc_layer_body, not last, last),
        grid=(N // BN,),
        in_specs=in_specs,
        out_specs=out_specs,
        out_shape=out_shape,
    )
    return fn(*args)


@jax.jit
def kernel(x, edge_index, edge_type, params):
    z64 = jnp.zeros((64, D), jnp.float32)
    zc = jnp.zeros((NPAD, RP), jnp.float32)

    npad_e = NW * EPW - E
    pad_i = jnp.arange(npad_e, dtype=jnp.int32)
    src = jnp.concatenate([edge_index[0], (pad_i * 197) % N]).reshape(NW, NCHK, CHUNK)
    dst = jnp.concatenate([edge_index[1], N + (pad_i % 16)]).reshape(NW, NCHK, CHUNK)
    et = jnp.concatenate([edge_type, pad_i % R]).reshape(NW, NCHK, CHUNK)
    ei = jnp.stack([et * N + src, dst, jnp.minimum(dst, N - 1), et], axis=2)

    sc_counts = _make_sc_counts()
    sc_agg = _make_sc_agg()

    (cnt,) = sc_counts(dst, et, zc)
    xw1 = _tc_xw(x, params[0])
    norm = _tc_norm(cnt)
    (p1,) = sc_agg(xw1.reshape(R * N, D), norm, ei, z64)
    h1, xw2 = _tc_layer(x, p1, params[0], params[1])
    (p2,) = sc_agg(xw2.reshape(R * N, D), norm, ei, z64)
    h2, xw3 = _tc_layer(h1, p2, params[1], params[2])
    (p3,) = sc_agg(xw3.reshape(R * N, D), norm, ei, z64)
    (h3,) = _tc_layer(h2, p3, params[2], None)
    return h3


# speedup vs baseline: 30.1364x; 1.0783x over previous
import functools

import jax
import jax.numpy as jnp
from jax import lax
from jax.experimental import pallas as pl
from jax.experimental.pallas import tpu as pltpu
from jax.experimental.pallas import tpu_sc as plsc

N = 10000
E = 320000
D = 128
R = 8
RP = 16
NB = 4

NC = 2
NS = 16
NW = NC * NS
CHUNK = 128
NCHK = 80
EPW = NCHK * CHUNK
NPAD = N + 16
RPW = NPAD // NS
BN = 1000


def _sc_counts_body(dst_ref, typ_ref, zc_ref, cnt_ref,
                    d2d, t2d, tbuf, oh0, oh1, cacc, sc0, sc1):
    c = lax.axis_index("c")
    s = lax.axis_index("s")
    wid = c * NS + s

    pltpu.sync_copy(dst_ref.at[wid], d2d)
    pltpu.sync_copy(typ_ref.at[wid], t2d)

    pltpu.sync_copy(zc_ref.at[pl.ds(s * RPW, RPW)],
                    cacc.at[pl.ds(s * RPW, RPW)])
    plsc.subcore_barrier()

    ones = jnp.ones((16,), jnp.float32)
    zv = jnp.zeros((16,), jnp.float32)
    iota = lax.iota(jnp.int32, 16)
    ohs = (oh0, oh1)
    scs = (sc0, sc1)

    def build(j, b):
        oh = ohs[b]
        for g in range(CHUNK // 16):
            sl = pl.ds(g * 16, 16)
            tbuf[sl] = t2d[j, sl]

        @plsc.parallel_loop(0, CHUNK, unroll=4)
        def _(e):
            tv = plsc.load_gather(tbuf, [jnp.full((16,), 0, jnp.int32) + e])
            oh[e, :] = jnp.where(iota == tv, ones, zv)

    def scat(j, b):
        pltpu.async_copy(ohs[b], cacc.at[d2d.at[j]], scs[b], add=True)

    def wait_scat(b):
        pltpu.make_async_copy(ohs[b], cacc.at[pl.ds(0, CHUNK)],
                              scs[b]).wait()

    build(0, 0)

    def pair(i, _):
        j = 2 * i
        scat(j, 0)
        build(j + 1, 1)
        scat(j + 1, 1)
        wait_scat(0)
        build(jnp.minimum(j + 2, NCHK - 1), 0)
        wait_scat(1)
        return 0
    lax.fori_loop(0, NCHK // 2, pair, 0)
    plsc.subcore_barrier()

    pltpu.sync_copy(cacc.at[pl.ds(s * RPW, RPW)],
                    cnt_ref.at[c, pl.ds(s * RPW, RPW)])


def _sc_agg_body(xw_ref, norm_ref, ei_ref, z64_ref, p_ref,
                 eb0, eb1, db0, db1, nb0, nb1, nr0, nr1, rb0, rb1, zbuf,
                 acc, sx0, sx1, sn0, sn1, ss0, ss1):
    c = lax.axis_index("c")
    s = lax.axis_index("s")
    wid = c * NS + s

    pltpu.sync_copy(z64_ref, zbuf)

    for k in range(RPW // 64):
        pltpu.sync_copy(zbuf, acc.at[pl.ds(s * RPW + k * 64, 64)])
    pltpu.sync_copy(zbuf.at[pl.ds(0, RPW % 64)],
                    acc.at[pl.ds(s * RPW + (RPW // 64) * 64, RPW % 64)])
    plsc.subcore_barrier()

    iota = lax.iota(jnp.int32, 16)
    ebs = (eb0, eb1)
    dbs = (db0, db1)
    nbs = (nb0, nb1)
    nrs = (nr0, nr1)
    rbs = (rb0, rb1)
    sxs = (sx0, sx1)
    sns = (sn0, sn1)
    sss = (ss0, ss1)

    def fire_ei(j, b):
        pltpu.sync_copy(ei_ref.at[wid, j], ebs[b])

    def fire(b):
        pltpu.async_copy(xw_ref.at[ebs[b].at[0]], rbs[b], sxs[b])
        pltpu.async_copy(norm_ref.at[ebs[b].at[2]], nrs[b], sns[b])

    def scale(b):
        eb, nb, nr, rb = ebs[b], nbs[b], nrs[b], rbs[b]
        pltpu.make_async_copy(xw_ref.at[pl.ds(0, CHUNK)], rb, sxs[b]).wait()
        pltpu.make_async_copy(norm_ref.at[pl.ds(0, CHUNK)], nr, sns[b]).wait()
        for g in range(CHUNK // 16):
            sl = pl.ds(g * 16, 16)
            nb[sl] = plsc.load_gather(nr, [g * 16 + iota, eb[3, sl]])

        @plsc.parallel_loop(0, CHUNK, unroll=4)
        def _(e):
            ns = plsc.load_gather(nb, [jnp.full((16,), 0, jnp.int32) + e])
            for q in range(D // 16):
                qs = pl.ds(q * 16, 16)
                rb[e, qs] = rb[e, qs] * ns

    def scat(b):
        for g in range(CHUNK // 16):
            sl = pl.ds(g * 16, 16)
            dbs[b][sl] = ebs[b][1, sl]
        pltpu.async_copy(rbs[b], acc.at[dbs[b]], sss[b], add=True)

    def wait_scat(b):
        pltpu.make_async_copy(rbs[b], acc.at[pl.ds(0, CHUNK)], sss[b]).wait()

    fire_ei(0, 0)
    fire(0)
    fire_ei(1, 1)
    fire(1)

    def pair(i, _):
        j = 2 * i
        scale(0)
        scat(0)
        fire_ei(jnp.minimum(j + 2, NCHK - 1), 0)
        scale(1)
        wait_scat(0)
        fire(0)
        scat(1)
        fire_ei(jnp.minimum(j + 3, NCHK - 1), 1)
        wait_scat(1)
        fire(1)
        return 0
    lax.fori_loop(0, NCHK // 2, pair, 0)

    pltpu.make_async_copy(xw_ref.at[pl.ds(0, CHUNK)], rb0, sx0).wait()
    pltpu.make_async_copy(norm_ref.at[pl.ds(0, CHUNK)], nr0, sn0).wait()
    pltpu.make_async_copy(xw_ref.at[pl.ds(0, CHUNK)], rb1, sx1).wait()
    pltpu.make_async_copy(norm_ref.at[pl.ds(0, CHUNK)], nr1, sn1).wait()
    plsc.subcore_barrier()

    pltpu.sync_copy(acc.at[pl.ds(s * RPW, RPW)],
                    p_ref.at[c, pl.ds(s * RPW, RPW)])


_SC_PARAMS = pltpu.CompilerParams(use_tc_tiling_on_sc=False,
                                  needs_layout_passes=False)
_SC_MESH = dict(core_axis_name="c", subcore_axis_name="s")


def _make_sc_counts():
    return pl.kernel(
        _sc_counts_body,
        out_type=(jax.ShapeDtypeStruct((NC, NPAD, RP), jnp.float32),),
        mesh=plsc.VectorSubcoreMesh(**_SC_MESH),
        scratch_types=(
            pltpu.VMEM((NCHK, CHUNK), jnp.int32),
            pltpu.VMEM((NCHK, CHUNK), jnp.int32),
            pltpu.VMEM((CHUNK,), jnp.int32),
            pltpu.VMEM((CHUNK, RP), jnp.float32),
            pltpu.VMEM((CHUNK, RP), jnp.float32),
            pltpu.VMEM_SHARED((NPAD, RP), jnp.float32),
            pltpu.SemaphoreType.DMA,
            pltpu.SemaphoreType.DMA,
        ),
        compiler_params=_SC_PARAMS,
    )


def _make_sc_agg():
    return pl.kernel(
        _sc_agg_body,
        out_type=(jax.ShapeDtypeStruct((NC, NPAD, D), jnp.float32),),
        mesh=plsc.VectorSubcoreMesh(**_SC_MESH),
        scratch_types=(
            pltpu.VMEM((4, CHUNK), jnp.int32),
            pltpu.VMEM((4, CHUNK), jnp.int32),
            pltpu.VMEM((CHUNK,), jnp.int32),
            pltpu.VMEM((CHUNK,), jnp.int32),
            pltpu.VMEM((CHUNK,), jnp.float32),
            pltpu.VMEM((CHUNK,), jnp.float32),
            pltpu.VMEM((CHUNK, RP), jnp.float32),
            pltpu.VMEM((CHUNK, RP), jnp.float32),
            pltpu.VMEM((CHUNK, D), jnp.float32),
            pltpu.VMEM((CHUNK, D), jnp.float32),
            pltpu.VMEM((64, D), jnp.float32),
            pltpu.VMEM_SHARED((NPAD, D), jnp.float32),
            pltpu.SemaphoreType.DMA,
            pltpu.SemaphoreType.DMA,
            pltpu.SemaphoreType.DMA,
            pltpu.SemaphoreType.DMA,
            pltpu.SemaphoreType.DMA,
            pltpu.SemaphoreType.DMA,
        ),
        compiler_params=_SC_PARAMS,
    )


def _tc_xw_body(x_ref, weight_ref, comp_ref, xw_ref):
    w = jnp.dot(comp_ref[...], weight_ref[...].reshape(NB, D * D),
                preferred_element_type=jnp.float32).reshape(R, D, D)
    x = x_ref[...]
    for r in range(R):
        xw_ref[r] = jnp.dot(x, w[r], preferred_element_type=jnp.float32)


def _tc_xw(x, p0):
    fn = pl.pallas_call(
        _tc_xw_body,
        grid=(N // BN,),
        in_specs=[
            pl.BlockSpec((BN, D), lambda i: (i, 0)),
            pl.BlockSpec((NB, D, D), lambda i: (0, 0, 0)),
            pl.BlockSpec((R, NB), lambda i: (0, 0)),
        ],
        out_specs=pl.BlockSpec((R, BN, D), lambda i: (0, i, 0)),
        out_shape=jax.ShapeDtypeStruct((R, N, D), jnp.float32),
    )
    return fn(x, p0['weight'], p0['comp'])


def _tc_norm_body(cnt_ref, norm_ref):
    cnt = cnt_ref[0] + cnt_ref[1]
    norm_ref[...] = 1.0 / jnp.maximum(cnt, 1.0)


def _tc_norm(cnt):
    fn = pl.pallas_call(
        _tc_norm_body,
        grid=(N // BN,),
        in_specs=[pl.BlockSpec((NC, BN, RP), lambda i: (0, i, 0))],
        out_specs=pl.BlockSpec((BN, RP), lambda i: (i, 0)),
        out_shape=jax.ShapeDtypeStruct((N, RP), jnp.float32),
    )
    return fn(cnt)


def _tc_layer_body(outer_relu, last, x_ref, p_ref, root_ref, bias_ref,
                   eps_ref, w1_ref, b1_ref, g_ref, bln_ref, w2_ref, b2_ref,
                   *rest):
    if last:
        (h_ref,) = rest
    else:
        nweight_ref, ncomp_ref, h_ref, xw_ref = rest
    x = x_ref[...]
    pblk = p_ref[...]
    agg = pblk[0] + pblk[1]
    out = agg + jnp.dot(x, root_ref[...], preferred_element_type=jnp.float32)
    out = out + bias_ref[...] + (1.0 + eps_ref[0, 0]) * x
    h = jnp.dot(out, w1_ref[...], preferred_element_type=jnp.float32) + b1_ref[...]
    mu = jnp.mean(h, axis=-1, keepdims=True)
    var = jnp.mean((h - mu) ** 2, axis=-1, keepdims=True)
    h = (h - mu) * lax.rsqrt(var + 1e-5) * g_ref[...] + bln_ref[...]
    h = jnp.maximum(h, 0.0)
    h = jnp.dot(h, w2_ref[...], preferred_element_type=jnp.float32) + b2_ref[...]
    if outer_relu:
        h = jnp.maximum(h, 0.0)
    h_ref[...] = h
    if not last:
        w = jnp.dot(ncomp_ref[...], nweight_ref[...].reshape(NB, D * D),
                    preferred_element_type=jnp.float32).reshape(R, D, D)
        for r in range(R):
            xw_ref[r] = jnp.dot(h, w[r], preferred_element_type=jnp.float32)


def _tc_layer(x, pp, p, pnext):
    last = pnext is None
    in_specs = [
        pl.BlockSpec((BN, D), lambda i: (i, 0)),
        pl.BlockSpec((NC, BN, D), lambda i: (0, i, 0)),
        pl.BlockSpec((D, D), lambda i: (0, 0)),
        pl.BlockSpec((1, D), lambda i: (0, 0)),
        pl.BlockSpec((1, 1), lambda i: (0, 0)),
        pl.BlockSpec((D, D), lambda i: (0, 0)),
        pl.BlockSpec((1, D), lambda i: (0, 0)),
        pl.BlockSpec((1, D), lambda i: (0, 0)),
        pl.BlockSpec((1, D), lambda i: (0, 0)),
        pl.BlockSpec((D, D), lambda i: (0, 0)),
        pl.BlockSpec((1, D), lambda i: (0, 0)),
    ]
    args = [x, pp, p['root'], p['bias'].reshape(1, D), p['eps'].reshape(1, 1),
            p['W1'], p['b1'].reshape(1, D), p['g'].reshape(1, D),
            p['bln'].reshape(1, D), p['W2'], p['b2'].reshape(1, D)]
    out_specs = [pl.BlockSpec((BN, D), lambda i: (i, 0))]
    out_shape = [jax.ShapeDtypeStruct((N, D), jnp.float32)]
    if not last:
        in_specs += [pl.BlockSpec((NB, D, D), lambda i: (0, 0, 0)),
                     pl.BlockSpec((R, NB), lambda i: (0, 0))]
        args += [pnext['weight'], pnext['comp']]
        out_specs.append(pl.BlockSpec((R, BN, D), lambda i: (0, i, 0)))
        out_shape.append(jax.ShapeDtypeStruct((R, N, D), jnp.float32))
    fn = pl.pallas_call(
        functools.partial(_tc_layer_body, not last, last),
        grid=(N // BN,),
        in_specs=in_specs,
        out_specs=out_specs,
        out_shape=out_shape,
    )
    return fn(*args)


@jax.jit
def kernel(x, edge_index, edge_type, params):
    z64 = jnp.zeros((64, D), jnp.float32)
    zc = jnp.zeros((NPAD, RP), jnp.float32)

    npad_e = NW * EPW - E
    pad_i = jnp.arange(npad_e, dtype=jnp.int32)
    src = jnp.concatenate([edge_index[0], (pad_i * 197) % N]).reshape(NW, NCHK, CHUNK)
    dst = jnp.concatenate([edge_index[1], N + (pad_i % 16)]).reshape(NW, NCHK, CHUNK)
    et = jnp.concatenate([edge_type, pad_i % R]).reshape(NW, NCHK, CHUNK)
    ei = jnp.stack([et * N + src, dst, jnp.minimum(dst, N - 1), et], axis=2)

    sc_counts = _make_sc_counts()
    sc_agg = _make_sc_agg()

    (cnt,) = sc_counts(dst, et, zc)
    xw1 = _tc_xw(x, params[0])
    norm = _tc_norm(cnt)
    (p1,) = sc_agg(xw1.reshape(R * N, D), norm, ei, z64)
    h1, xw2 = _tc_layer(x, p1, params[0], params[1])
    (p2,) = sc_agg(xw2.reshape(R * N, D), norm, ei, z64)
    h2, xw3 = _tc_layer(h1, p2, params[1], params[2])
    (p3,) = sc_agg(xw3.reshape(R * N, D), norm, ei, z64)
    (h3,) = _tc_layer(h2, p3, params[2], None)
    return h3
